# baseline (speedup 1.0000x reference)
import functools

import jax
import jax.numpy as jnp
from jax import lax
from jax.experimental import pallas as pl
from jax.experimental.pallas import tpu as pltpu
from jax.experimental.pallas import tpu_sc as plsc

NUM_EXPERTS = 64
TOP_K = 8
B = 32768

_NC = 2
_NS = 16
_NW = _NC * _NS
_TOK_W = B // _NW
_NCHUNK = 4
_CHUNK = _TOK_W // _NCHUNK


@functools.partial(
    pl.kernel,
    out_type=(
        jax.ShapeDtypeStruct((TOP_K, B), jnp.float32),
        jax.ShapeDtypeStruct((TOP_K, B), jnp.int32),
    ),
    mesh=plsc.VectorSubcoreMesh(core_axis_name="c", subcore_axis_name="s"),
    compiler_params=pltpu.CompilerParams(needs_layout_passes=False),
    scratch_types=[
        pltpu.VMEM((_CHUNK, NUM_EXPERTS), jnp.float32),
        pltpu.VMEM((_CHUNK, NUM_EXPERTS), jnp.float32),
        pltpu.VMEM((TOP_K, _TOK_W), jnp.float32),
        pltpu.VMEM((TOP_K, _TOK_W), jnp.int32),
        pltpu.SemaphoreType.DMA,
        pltpu.SemaphoreType.DMA,
    ],
)
def _router(logits_hbm, vals_hbm, idx_hbm, lbufa, lbufb, vbuf, ibuf, sema, semb):
    wid = lax.axis_index("s") * _NC + lax.axis_index("c")
    base = wid * _TOK_W
    bufs = (lbufa, lbufb)
    sems = (sema, semb)

    lane = lax.iota(jnp.int32, 16)
    mask8 = lane < 8
    row8 = lane & 7
    rot8 = lane ^ 8

    def start(chunk):
        return pltpu.async_copy(
            logits_hbm.at[pl.ds(base + chunk * _CHUNK, _CHUNK)],
            bufs[chunk % 2], sems[chunk % 2])

    pending = start(0)
    for chunk in range(_NCHUNK):
        pending.wait()
        if chunk + 1 < _NCHUNK:
            pending = start(chunk + 1)
        lbuf = bufs[chunk % 2]

        @plsc.parallel_loop(0, _CHUNK, unroll=8)
        def _token(t, chunk=chunk, lbuf=lbuf):
            l0 = lbuf[t, pl.ds(0, 16)]
            l1 = lbuf[t, pl.ds(16, 16)]
            l2 = lbuf[t, pl.ds(32, 16)]
            l3 = lbuf[t, pl.ds(48, 16)]
            mc = plsc.cummax(jnp.maximum(jnp.maximum(l0, l1),
                                         jnp.maximum(l2, l3)))
            m = mc[jnp.full((16,), 15, jnp.int32)]
            e0 = jnp.exp(l0 - m)
            e1 = jnp.exp(l1 - m)
            e2 = jnp.exp(l2 - m)
            e3 = jnp.exp(l3 - m)
            zc = plsc.cumsum((e0 + e1) + (e2 + e3))
            z = zc[jnp.full((16,), 15, jnp.int32)]

            def pack(e, j):
                bits = lax.bitcast_convert_type(e, jnp.uint32)
                bits = jnp.clip(bits, jnp.uint32(0x3B800000),
                                jnp.uint32(0x3F7FFFFF))
                tie = (63 - 16 * j - lane).astype(jnp.uint32)
                span = (bits - jnp.uint32(0x3B800000)) >> 1
                return (span << 6) | tie

            k0, k1 = pack(e0, 0), pack(e1, 1)
            k2, k3 = pack(e2, 2), pack(e3, 3)

            n0 = lax.sort(~k0)
            s1 = lax.sort(k1)
            n2 = lax.sort(~k2)
            s3 = lax.sort(k3)

            c01 = jnp.where(mask8, ~n0, s1)
            d01 = jnp.maximum(c01, c01[rot8])
            c23 = jnp.where(mask8, ~n2, s3)
            d23 = jnp.maximum(c23, c23[rot8])

            fk = ~lax.sort(~jnp.where(mask8, d01, d23))
            idx = 63 - lax.bitcast_convert_type(fk & 63, jnp.int32)

            col = jnp.full((16,), t, jnp.int32)
            ge = jnp.exp(plsc.load_gather(lbuf, [col, idx & 63]) - m)
            sc = plsc.cumsum(jnp.where(mask8, ge, 0.0))
            s8 = sc[jnp.full((16,), 15, jnp.int32)]
            r = 1.0 / (s8 + 1e-6 * z)

            ocol = jnp.full((16,), chunk * _CHUNK + t, jnp.int32)
            plsc.store_scatter(vbuf, [row8, ocol], ge * r, mask=mask8)
            plsc.store_scatter(ibuf, [row8, ocol], idx, mask=mask8)

    pltpu.sync_copy(vbuf, vals_hbm.at[:, pl.ds(base, _TOK_W)])
    pltpu.sync_copy(ibuf, idx_hbm.at[:, pl.ds(base, _TOK_W)])


def kernel(logits, noise_std, training):
    del noise_std, training
    vals, idx = _router(logits)
    return vals.T, idx.T


# speedup vs baseline: 1.3684x; 1.3684x over previous
import functools

import jax
import jax.numpy as jnp
from jax import lax
from jax.experimental import pallas as pl
from jax.experimental.pallas import tpu as pltpu
from jax.experimental.pallas import tpu_sc as plsc

NUM_EXPERTS = 64
TOP_K = 8
B = 32768

_NC = 2
_NS = 16
_NW = _NC * _NS
_TOK_W = B // _NW
_NCHUNK = 4
_CHUNK = _TOK_W // _NCHUNK


@functools.partial(
    pl.kernel,
    out_type=(
        jax.ShapeDtypeStruct((TOP_K, B), jnp.float32),
        jax.ShapeDtypeStruct((TOP_K, B), jnp.int32),
    ),
    mesh=plsc.VectorSubcoreMesh(core_axis_name="c", subcore_axis_name="s"),
    compiler_params=pltpu.CompilerParams(needs_layout_passes=False),
    scratch_types=[
        pltpu.VMEM((_CHUNK, NUM_EXPERTS), jnp.float32),
        pltpu.VMEM((_CHUNK, NUM_EXPERTS), jnp.float32),
        pltpu.VMEM((TOP_K, _TOK_W), jnp.float32),
        pltpu.VMEM((TOP_K, _TOK_W), jnp.int32),
        pltpu.SemaphoreType.DMA,
        pltpu.SemaphoreType.DMA,
    ],
)
def _router(logits_hbm, vals_hbm, idx_hbm, lbufa, lbufb, vbuf, ibuf, sema, semb):
    wid = lax.axis_index("s") * _NC + lax.axis_index("c")
    base = wid * _TOK_W
    bufs = (lbufa, lbufb)
    sems = (sema, semb)

    lane = lax.iota(jnp.int32, 16)
    mask8 = lane < 8
    row8 = lane & 7
    rot8 = lane ^ 8

    def half_clean(ka, va, kb, vb):
        ck = jnp.where(mask8, ka, kb)
        cv = jnp.where(mask8, va, vb)
        rk = ck[rot8]
        rv = cv[rot8]
        ge = ck >= rk
        return jnp.where(ge, ck, rk), jnp.where(ge, cv, rv)

    def start(chunk):
        return pltpu.async_copy(
            logits_hbm.at[pl.ds(base + chunk * _CHUNK, _CHUNK)],
            bufs[chunk % 2], sems[chunk % 2])

    pending = start(0)
    for chunk in range(_NCHUNK):
        pending.wait()
        if chunk + 1 < _NCHUNK:
            pending = start(chunk + 1)
        lbuf = bufs[chunk % 2]

        @plsc.parallel_loop(0, _CHUNK, unroll=8)
        def _token(t, chunk=chunk, lbuf=lbuf):
            e0 = jnp.exp(lbuf[t, pl.ds(0, 16)])
            e1 = jnp.exp(lbuf[t, pl.ds(16, 16)])
            e2 = jnp.exp(lbuf[t, pl.ds(32, 16)])
            e3 = jnp.exp(lbuf[t, pl.ds(48, 16)])
            zc = plsc.cumsum((e0 + e1) + (e2 + e3))
            z = zc[jnp.full((16,), 15, jnp.int32)]

            k0, v0 = plsc.sort_key_val(e0, lane, descending=True)
            k1, v1 = plsc.sort_key_val(e1, lane + 16, descending=False)
            k2, v2 = plsc.sort_key_val(e2, lane + 32, descending=True)
            k3, v3 = plsc.sort_key_val(e3, lane + 48, descending=False)

            d01k, d01v = half_clean(k0, v0, k1, v1)
            d23k, d23v = half_clean(k2, v2, k3, v3)

            fk, fv = plsc.sort_key_val(
                jnp.where(mask8, d01k, d23k), jnp.where(mask8, d01v, d23v),
                descending=True)

            s8 = plsc.cumsum(fk)[jnp.full((16,), TOP_K - 1, jnp.int32)]
            r = 1.0 / (s8 + 1e-6 * z)

            col = jnp.full((16,), chunk * _CHUNK + t, jnp.int32)
            plsc.store_scatter(vbuf, [row8, col], fk * r, mask=mask8)
            plsc.store_scatter(ibuf, [row8, col], fv, mask=mask8)

    pltpu.sync_copy(vbuf, vals_hbm.at[:, pl.ds(base, _TOK_W)])
    pltpu.sync_copy(ibuf, idx_hbm.at[:, pl.ds(base, _TOK_W)])


def kernel(logits, noise_std, training):
    del noise_std, training
    vals, idx = _router(logits)
    return vals.T, idx.T
